# baseline (speedup 1.0000x reference)
import jax
import jax.numpy as jnp
import numpy as np
from jax import lax
from jax.experimental import pallas as pl
from jax.experimental.pallas import tpu as pltpu
from jax.experimental.pallas import tpu_sc as plsc

NC = 2
NS = 16
NW = NC * NS
LANES = 16
L = 200
NF = 16
CH = 128
RB = 800
SHIFT = L - 128
MAGIC = 12582912.0


def _fmt_body(in_ref, a_ref, b_ref):
    x = in_ref[...]
    a_ref[...] = x[:, 0:128]
    x256 = jnp.pad(x, ((0, 0), (0, 256 - L)))
    b_ref[...] = pltpu.roll(x256, 256 - SHIFT, 1)[:, 0:128]


def _sc_body(users_h, items_h, song_h, uf_h, if_h, ub_h, ib_h, rata_h, ratb_h,
             p_h, m_h,
             uidx, iidx, sidx, ufv, ifv, ubv, ibv, ratv, pv, mv, sem):
    wid = lax.axis_index("s") * NC + lax.axis_index("c")
    lane = lax.iota(jnp.int32, LANES)
    fzero = jnp.zeros((LANES,), jnp.float32)
    n_chunks = users_h.shape[0] // (NW * CH)
    for chunk in range(n_chunks):
        base = wid * (n_chunks * CH) + chunk * CH
        pltpu.sync_copy(users_h.at[pl.ds(base, CH)], uidx)
        pltpu.sync_copy(items_h.at[pl.ds(base, CH)], iidx)
        pltpu.sync_copy(song_h.at[pl.ds(base, CH)], sidx)

        cps = [
            pltpu.async_copy(rata_h.at[uidx], ratv.at[0], sem),
            pltpu.async_copy(ratb_h.at[uidx], ratv.at[1], sem),
            pltpu.async_copy(uf_h.at[uidx], ufv, sem),
            pltpu.async_copy(if_h.at[iidx], ifv, sem),
            pltpu.async_copy(ub_h.at[uidx], ubv, sem),
            pltpu.async_copy(ib_h.at[iidx], ibv, sem),
        ]
        for cp in cps:
            cp.wait()

        def group(g, carry):
            s0 = g * LANES
            idx16 = lane + s0
            dot = fzero
            for d in range(NF):
                dv = jnp.full((LANES,), d, jnp.int32)
                ufd = plsc.load_gather(ufv, [idx16, dv])
                ifd = plsc.load_gather(ifv, [idx16, dv])
                dot = dot + ufd * ifd
            izero = jnp.zeros((LANES,), jnp.int32)
            ub = plsc.load_gather(ubv, [idx16, izero])
            ib = plsc.load_gather(ibv, [idx16, izero])
            p = ub + ib + dot
            t = jnp.clip((p + MAGIC) - MAGIC, np.float32(0.0), np.float32(4.0))
            c0 = t + np.float32(1.5)

            def pos_body(posi, macc):
                k = posi >> 7
                vals = plsc.load_gather(
                    ratv, [jnp.full((LANES,), k, jnp.int32), idx16,
                           jnp.full((LANES,), posi - k * SHIFT, jnp.int32)])
                return macc + jnp.clip(c0 - vals, 0.0, 1.0)

            mass = lax.fori_loop(0, L, pos_body, fzero, unroll=8)
            si16 = sidx[pl.ds(s0, LANES)]
            ks = si16 >> 7
            old_r = plsc.load_gather(ratv, [ks, idx16, si16 - ks * SHIFT])
            mass = mass - jnp.clip(c0 - old_r, 0.0, 1.0)
            q = (p - np.float32(1.0)) / np.float32(0.8)
            qi = q.astype(jnp.int32)
            fl = qi - jnp.where(qi.astype(jnp.float32) > q, 1, 0)
            cp_ = jnp.clip(fl, 0, 4).astype(jnp.float32)
            in_range = jnp.where((p >= 1.0) & (p <= 5.0),
                                 np.float32(1.0), np.float32(0.0))
            mass = mass + in_range * jnp.clip(t + np.float32(0.5) - cp_, 0.0, 1.0)
            pv[pl.ds(s0, LANES)] = p
            mv[pl.ds(s0, LANES)] = mass
            return carry

        lax.fori_loop(0, CH // LANES, group, 0)
        pltpu.sync_copy(pv, p_h.at[pl.ds(base, CH)])
        pltpu.sync_copy(mv, m_h.at[pl.ds(base, CH)])


def kernel(users, items, user_factors, item_factors, user_biases, item_biases,
           ratings_by_user, song_index):
    users = users.astype(jnp.int32)
    items = items.astype(jnp.int32)
    song_index = song_index.astype(jnp.int32)
    n_users = user_factors.shape[0]
    B = users.shape[0]

    fmt = pl.pallas_call(
        _fmt_body,
        grid=(n_users // RB,),
        in_specs=[pl.BlockSpec((RB, L), lambda i: (i, 0))],
        out_specs=[pl.BlockSpec((RB, 128), lambda i: (i, 0)),
                   pl.BlockSpec((RB, 128), lambda i: (i, 0))],
        out_shape=[jax.ShapeDtypeStruct((n_users, 128), jnp.float32),
                   jax.ShapeDtypeStruct((n_users, 128), jnp.float32)],
    )
    rat_a, rat_b = fmt(ratings_by_user)

    mesh = plsc.VectorSubcoreMesh(core_axis_name="c", subcore_axis_name="s")
    run = pl.kernel(
        _sc_body,
        mesh=mesh,
        out_type=(jax.ShapeDtypeStruct((B,), jnp.float32),
                  jax.ShapeDtypeStruct((B,), jnp.float32)),
        compiler_params=pltpu.CompilerParams(
            needs_layout_passes=False, use_tc_tiling_on_sc=False),
        scratch_types=[
            pltpu.VMEM((CH,), jnp.int32),
            pltpu.VMEM((CH,), jnp.int32),
            pltpu.VMEM((CH,), jnp.int32),
            pltpu.VMEM((CH, NF), jnp.float32),
            pltpu.VMEM((CH, NF), jnp.float32),
            pltpu.VMEM((CH, 1), jnp.float32),
            pltpu.VMEM((CH, 1), jnp.float32),
            pltpu.VMEM((2, CH, 128), jnp.float32),
            pltpu.VMEM((CH,), jnp.float32),
            pltpu.VMEM((CH,), jnp.float32),
            pltpu.SemaphoreType.DMA,
        ],
    )
    p, mass = run(users, items, song_index, user_factors, item_factors,
                  user_biases, item_biases, rat_a, rat_b)
    return jnp.stack((p, mass), axis=1)[:, :, None]


# speedup vs baseline: 2.2132x; 2.2132x over previous
import jax
import jax.numpy as jnp
import numpy as np
from jax import lax
from jax.experimental import pallas as pl
from jax.experimental.pallas import tpu as pltpu
from jax.experimental.pallas import tpu_sc as plsc

NC = 2
NS = 16
NW = NC * NS
LANES = 16
L = 200
NF = 16
CH = 128
RB = 800
SHIFT = L - 128
UBL = 0
UFL = 1
MAGIC = 12582912.0


def _fmt_body(rat_ref, ub_ref, uf_ref, a_ref, b_ref):
    x = rat_ref[...]
    a_ref[...] = x[:, 0:128]
    x256 = jnp.pad(x, ((0, 0), (0, 256 - L)))
    b = pltpu.roll(x256, 256 - SHIFT, 1)[:, 0:128]
    packed = jnp.pad(jnp.concatenate([ub_ref[...], uf_ref[...]], axis=1),
                     ((0, 0), (0, 128 - NF - 1)))
    lanes = jax.lax.broadcasted_iota(jnp.int32, b.shape, 1)
    b_ref[...] = jnp.where(lanes <= NF, packed, b)


def _sc_body(users_h, items_h, song_h, if_h, ib_h, rata_h, ratb_h,
             p_h, m_h,
             uidx, iidx, sidx, ifv, ibv, ratv, pv, mv, sem):
    wid = lax.axis_index("s") * NC + lax.axis_index("c")
    lane = lax.iota(jnp.int32, LANES)
    fzero = jnp.zeros((LANES,), jnp.float32)
    ione = jnp.full((LANES,), 1, jnp.int32)
    n_chunks = users_h.shape[0] // (NW * CH)
    for chunk in range(n_chunks):
        base = wid * (n_chunks * CH) + chunk * CH
        pltpu.sync_copy(users_h.at[pl.ds(base, CH)], uidx)
        pltpu.sync_copy(items_h.at[pl.ds(base, CH)], iidx)
        pltpu.sync_copy(song_h.at[pl.ds(base, CH)], sidx)

        cps = [
            pltpu.async_copy(rata_h.at[uidx], ratv.at[0], sem),
            pltpu.async_copy(ratb_h.at[uidx], ratv.at[1], sem),
            pltpu.async_copy(if_h.at[iidx], ifv, sem),
            pltpu.async_copy(ib_h.at[iidx], ibv, sem),
        ]
        for cp in cps:
            cp.wait()

        def group(g, carry):
            s0 = g * LANES
            idx16 = lane + s0
            dot = fzero
            for d in range(NF):
                ufd = plsc.load_gather(
                    ratv, [ione, idx16, jnp.full((LANES,), UFL + d, jnp.int32)])
                ifd = plsc.load_gather(
                    ifv, [idx16, jnp.full((LANES,), d, jnp.int32)])
                dot = dot + ufd * ifd
            ub = plsc.load_gather(
                ratv, [ione, idx16, jnp.full((LANES,), UBL, jnp.int32)])
            ib = ibv[pl.ds(s0, LANES)]
            p = ub + ib + dot
            t = jnp.clip((p + MAGIC) - MAGIC, np.float32(0.0), np.float32(4.0))
            c0 = t + np.float32(1.5)

            def pos_body(posi, macc):
                k = posi >> 7
                vals = plsc.load_gather(
                    ratv, [jnp.full((LANES,), k, jnp.int32), idx16,
                           jnp.full((LANES,), posi - k * SHIFT, jnp.int32)])
                return macc + jnp.clip(c0 - vals, 0.0, 1.0)

            mass = lax.fori_loop(0, L, pos_body, fzero, unroll=8)
            si16 = sidx[pl.ds(s0, LANES)]
            ks = si16 >> 7
            old_r = plsc.load_gather(ratv, [ks, idx16, si16 - ks * SHIFT])
            mass = mass - jnp.clip(c0 - old_r, 0.0, 1.0)
            q = (p - np.float32(1.0)) / np.float32(0.8)
            qi = q.astype(jnp.int32)
            fl = qi - jnp.where(qi.astype(jnp.float32) > q, 1, 0)
            cp_ = jnp.clip(fl, 0, 4).astype(jnp.float32)
            in_range = jnp.where((p >= 1.0) & (p <= 5.0),
                                 np.float32(1.0), np.float32(0.0))
            mass = mass + in_range * jnp.clip(t + np.float32(0.5) - cp_, 0.0, 1.0)
            pv[pl.ds(s0, LANES)] = p
            mv[pl.ds(s0, LANES)] = mass
            return carry

        lax.fori_loop(0, CH // LANES, group, 0)
        pltpu.sync_copy(pv, p_h.at[pl.ds(base, CH)])
        pltpu.sync_copy(mv, m_h.at[pl.ds(base, CH)])


def kernel(users, items, user_factors, item_factors, user_biases, item_biases,
           ratings_by_user, song_index):
    users = users.astype(jnp.int32)
    items = items.astype(jnp.int32)
    song_index = song_index.astype(jnp.int32)
    n_users = user_factors.shape[0]
    ib1 = jnp.sum(item_biases, axis=1)
    B = users.shape[0]

    fmt = pl.pallas_call(
        _fmt_body,
        grid=(n_users // RB,),
        in_specs=[pl.BlockSpec((RB, L), lambda i: (i, 0)),
                  pl.BlockSpec((RB, 1), lambda i: (i, 0)),
                  pl.BlockSpec((RB, NF), lambda i: (i, 0))],
        out_specs=[pl.BlockSpec((RB, 128), lambda i: (i, 0)),
                   pl.BlockSpec((RB, 128), lambda i: (i, 0))],
        out_shape=[jax.ShapeDtypeStruct((n_users, 128), jnp.float32),
                   jax.ShapeDtypeStruct((n_users, 128), jnp.float32)],
    )
    rat_a, rat_b = fmt(ratings_by_user, user_biases, user_factors)

    mesh = plsc.VectorSubcoreMesh(core_axis_name="c", subcore_axis_name="s")
    run = pl.kernel(
        _sc_body,
        mesh=mesh,
        out_type=(jax.ShapeDtypeStruct((B,), jnp.float32),
                  jax.ShapeDtypeStruct((B,), jnp.float32)),
        compiler_params=pltpu.CompilerParams(
            needs_layout_passes=False, use_tc_tiling_on_sc=False),
        scratch_types=[
            pltpu.VMEM((CH,), jnp.int32),
            pltpu.VMEM((CH,), jnp.int32),
            pltpu.VMEM((CH,), jnp.int32),
            pltpu.VMEM((CH, NF), jnp.float32),
            pltpu.VMEM((CH,), jnp.float32),
            pltpu.VMEM((2, CH, 128), jnp.float32),
            pltpu.VMEM((CH,), jnp.float32),
            pltpu.VMEM((CH,), jnp.float32),
            pltpu.SemaphoreType.DMA,
        ],
    )
    p, mass = run(users, items, song_index, item_factors, ib1, rat_a, rat_b)
    return jnp.stack((p, mass), axis=1)[:, :, None]


# speedup vs baseline: 2.3116x; 1.0445x over previous
import jax
import jax.numpy as jnp
import numpy as np
from jax import lax
from jax.experimental import pallas as pl
from jax.experimental.pallas import tpu as pltpu
from jax.experimental.pallas import tpu_sc as plsc

NC = 2
NS = 16
NW = NC * NS
LANES = 16
L = 200
NF = 16
CH = 128
RB = 800
SHIFT = L - 128
UBL = 0
UFL = 1
MAGIC = 12582912.0


def _fmt_body(rat_ref, a_ref, b_ref):
    x = rat_ref[...]
    a_ref[...] = x[:, 0:128]
    x256 = jnp.pad(x, ((0, 0), (0, 256 - L)))
    b_ref[...] = pltpu.roll(x256, 256 - SHIFT, 1)[:, 0:128]


def _sc_body(users_h, items_h, song_h, uf_h, if_h, ub_h, ib_h, rata_h, ratb_h,
             p_h, m_h,
             uidx, iidx, sidx, ufv, ifv, ubv, ibv, ratv, pv, mv, sem):
    wid = lax.axis_index("s") * NC + lax.axis_index("c")
    lane = lax.iota(jnp.int32, LANES)
    fzero = jnp.zeros((LANES,), jnp.float32)
    ione = jnp.full((LANES,), 1, jnp.int32)
    n_chunks = users_h.shape[0] // (NW * CH)
    for chunk in range(n_chunks):
        base = wid * (n_chunks * CH) + chunk * CH
        pltpu.sync_copy(users_h.at[pl.ds(base, CH)], uidx)
        pltpu.sync_copy(items_h.at[pl.ds(base, CH)], iidx)
        pltpu.sync_copy(song_h.at[pl.ds(base, CH)], sidx)

        cps = [
            pltpu.async_copy(rata_h.at[uidx], ratv.at[0], sem),
            pltpu.async_copy(ratb_h.at[uidx], ratv.at[1], sem),
            pltpu.async_copy(uf_h.at[uidx], ufv, sem),
            pltpu.async_copy(if_h.at[iidx], ifv, sem),
            pltpu.async_copy(ub_h.at[uidx], ubv, sem),
            pltpu.async_copy(ib_h.at[iidx], ibv, sem),
        ]
        for cp in cps:
            cp.wait()

        def group(g, carry):
            s0 = g * LANES
            idx16 = lane + s0
            dot = fzero
            for d in range(NF):
                dv = jnp.full((LANES,), d, jnp.int32)
                ufd = plsc.load_gather(ufv, [idx16, dv])
                ifd = plsc.load_gather(ifv, [idx16, dv])
                dot = dot + ufd * ifd
            p = ubv[pl.ds(s0, LANES)] + ibv[pl.ds(s0, LANES)] + dot
            t = jnp.clip((p + MAGIC) - MAGIC, np.float32(0.0), np.float32(4.0))
            c0 = t + np.float32(1.5)

            def pos_body(posi, macc):
                k = posi >> 7
                vals = plsc.load_gather(
                    ratv, [jnp.full((LANES,), k, jnp.int32), idx16,
                           jnp.full((LANES,), posi - k * SHIFT, jnp.int32)])
                return macc + jnp.clip(c0 - vals, 0.0, 1.0)

            mass = lax.fori_loop(0, L, pos_body, fzero, unroll=8)
            si16 = sidx[pl.ds(s0, LANES)]
            ks = si16 >> 7
            old_r = plsc.load_gather(ratv, [ks, idx16, si16 - ks * SHIFT])
            mass = mass - jnp.clip(c0 - old_r, 0.0, 1.0)
            q = (p - np.float32(1.0)) / np.float32(0.8)
            qi = q.astype(jnp.int32)
            fl = qi - jnp.where(qi.astype(jnp.float32) > q, 1, 0)
            cp_ = jnp.clip(fl, 0, 4).astype(jnp.float32)
            in_range = jnp.where((p >= 1.0) & (p <= 5.0),
                                 np.float32(1.0), np.float32(0.0))
            mass = mass + in_range * jnp.clip(t + np.float32(0.5) - cp_, 0.0, 1.0)
            pv[pl.ds(s0, LANES)] = p
            mv[pl.ds(s0, LANES)] = mass
            return carry

        lax.fori_loop(0, CH // LANES, group, 0)
        pltpu.sync_copy(pv, p_h.at[pl.ds(base, CH)])
        pltpu.sync_copy(mv, m_h.at[pl.ds(base, CH)])


def kernel(users, items, user_factors, item_factors, user_biases, item_biases,
           ratings_by_user, song_index):
    users = users.astype(jnp.int32)
    items = items.astype(jnp.int32)
    song_index = song_index.astype(jnp.int32)
    n_users = user_factors.shape[0]
    ub1 = jnp.sum(user_biases, axis=1)
    ib1 = jnp.sum(item_biases, axis=1)
    B = users.shape[0]

    fmt = pl.pallas_call(
        _fmt_body,
        grid=(n_users // RB,),
        in_specs=[pl.BlockSpec((RB, L), lambda i: (i, 0))],
        out_specs=[pl.BlockSpec((RB, 128), lambda i: (i, 0)),
                   pl.BlockSpec((RB, 128), lambda i: (i, 0))],
        out_shape=[jax.ShapeDtypeStruct((n_users, 128), jnp.float32),
                   jax.ShapeDtypeStruct((n_users, 128), jnp.float32)],
    )
    rat_a, rat_b = fmt(ratings_by_user)

    mesh = plsc.VectorSubcoreMesh(core_axis_name="c", subcore_axis_name="s")
    run = pl.kernel(
        _sc_body,
        mesh=mesh,
        out_type=(jax.ShapeDtypeStruct((B,), jnp.float32),
                  jax.ShapeDtypeStruct((B,), jnp.float32)),
        compiler_params=pltpu.CompilerParams(
            needs_layout_passes=False, use_tc_tiling_on_sc=False),
        scratch_types=[
            pltpu.VMEM((CH,), jnp.int32),
            pltpu.VMEM((CH,), jnp.int32),
            pltpu.VMEM((CH,), jnp.int32),
            pltpu.VMEM((CH, NF), jnp.float32),
            pltpu.VMEM((CH, NF), jnp.float32),
            pltpu.VMEM((CH,), jnp.float32),
            pltpu.VMEM((CH,), jnp.float32),
            pltpu.VMEM((2, CH, 128), jnp.float32),
            pltpu.VMEM((CH,), jnp.float32),
            pltpu.VMEM((CH,), jnp.float32),
            pltpu.SemaphoreType.DMA,
        ],
    )
    p, mass = run(users, items, song_index, user_factors, item_factors,
                  ub1, ib1, rat_a, rat_b)
    return jnp.stack((p, mass), axis=1)[:, :, None]


# speedup vs baseline: 2.4884x; 1.0765x over previous
import jax
import jax.numpy as jnp
import numpy as np
from jax import lax
from jax.experimental import pallas as pl
from jax.experimental.pallas import tpu as pltpu
from jax.experimental.pallas import tpu_sc as plsc

NC = 2
NS = 16
NW = NC * NS
LANES = 16
L = 200
NF = 16
CH = 128
RB = 800
SHIFT = L - 128
UBL = 0
UFL = 1
MAGIC = 12582912.0


def _fmt_body(rat_ref, a_ref, b_ref):
    x = rat_ref[...]
    a_ref[...] = x[:, 0:128]
    x256 = jnp.pad(x, ((0, 0), (0, 256 - L)))
    b_ref[...] = pltpu.roll(x256, 256 - SHIFT, 1)[:, 0:128]


def _sc_body(users_h, items_h, song_h, uf_h, if_h, ub_h, ib_h, rata_h, ratb_h,
             p_h, m_h,
             uidx, iidx, sidx, ufv, ifv, ubv, ibv, ratv, pv, mv, sem):
    wid = lax.axis_index("s") * NC + lax.axis_index("c")
    lane = lax.iota(jnp.int32, LANES)
    fzero = jnp.zeros((LANES,), jnp.float32)
    ione = jnp.full((LANES,), 1, jnp.int32)
    n_chunks = users_h.shape[0] // (NW * CH)
    for chunk in range(n_chunks):
        base = wid * (n_chunks * CH) + chunk * CH
        pltpu.sync_copy(users_h.at[pl.ds(base, CH)], uidx)
        pltpu.sync_copy(items_h.at[pl.ds(base, CH)], iidx)
        pltpu.sync_copy(song_h.at[pl.ds(base, CH)], sidx)

        cps = [
            pltpu.async_copy(rata_h.at[uidx], ratv.at[0], sem),
            pltpu.async_copy(ratb_h.at[uidx], ratv.at[1], sem),
            pltpu.async_copy(uf_h.at[uidx], ufv, sem),
            pltpu.async_copy(if_h.at[iidx], ifv, sem),
            pltpu.async_copy(ub_h.at[uidx], ubv, sem),
            pltpu.async_copy(ib_h.at[iidx], ibv, sem),
        ]
        for cp in cps:
            cp.wait()

        def group(g, carry):
            s0 = g * LANES
            idx16 = lane + s0
            dot = fzero
            for d in range(NF):
                dv = jnp.full((LANES,), d, jnp.int32)
                ufd = plsc.load_gather(ufv, [idx16, dv])
                ifd = plsc.load_gather(ifv, [idx16, dv])
                dot = dot + ufd * ifd
            p = ubv[pl.ds(s0, LANES)] + ibv[pl.ds(s0, LANES)] + dot
            t = jnp.clip((p + MAGIC) - MAGIC, np.float32(0.0), np.float32(4.0))
            c0 = t + np.float32(1.5)

            def pos_body(posi, macc):
                k = posi >> 7
                vals = plsc.load_gather(
                    ratv, [jnp.full((LANES,), k, jnp.int32), idx16,
                           jnp.full((LANES,), posi - k * SHIFT, jnp.int32)])
                return macc + jnp.clip(c0 - vals, 0.0, 1.0)

            mass = lax.fori_loop(0, L, pos_body, fzero, unroll=8)
            si16 = sidx[pl.ds(s0, LANES)]
            ks = si16 >> 7
            old_r = plsc.load_gather(ratv, [ks, idx16, si16 - ks * SHIFT])
            mass = mass - jnp.clip(c0 - old_r, 0.0, 1.0)
            q = (p - np.float32(1.0)) / np.float32(0.8)
            qi = q.astype(jnp.int32)
            fl = qi - jnp.where(qi.astype(jnp.float32) > q, 1, 0)
            cp_ = jnp.clip(fl, 0, 4).astype(jnp.float32)
            in_range = jnp.where((p >= 1.0) & (p <= 5.0),
                                 np.float32(1.0), np.float32(0.0))
            mass = mass + in_range * jnp.clip(t + np.float32(0.5) - cp_, 0.0, 1.0)
            pv[pl.ds(s0, LANES)] = p
            mv[pl.ds(s0, LANES)] = mass
            return carry

        lax.fori_loop(0, CH // LANES, group, 0)
        pltpu.sync_copy(pv, p_h.at[pl.ds(base, CH)])
        pltpu.sync_copy(mv, m_h.at[pl.ds(base, CH)])


def kernel(users, items, user_factors, item_factors, user_biases, item_biases,
           ratings_by_user, song_index):
    users = users.astype(jnp.int32)
    items = items.astype(jnp.int32)
    song_index = song_index.astype(jnp.int32)
    n_users = user_factors.shape[0]
    ub1 = jnp.sum(user_biases, axis=1)
    ib1 = jnp.sum(item_biases, axis=1)
    B = users.shape[0]

    rat_a = ratings_by_user[:, 0:128]
    rat_b = ratings_by_user[:, SHIFT:L]

    mesh = plsc.VectorSubcoreMesh(core_axis_name="c", subcore_axis_name="s")
    run = pl.kernel(
        _sc_body,
        mesh=mesh,
        out_type=(jax.ShapeDtypeStruct((B,), jnp.float32),
                  jax.ShapeDtypeStruct((B,), jnp.float32)),
        compiler_params=pltpu.CompilerParams(
            needs_layout_passes=False, use_tc_tiling_on_sc=False),
        scratch_types=[
            pltpu.VMEM((CH,), jnp.int32),
            pltpu.VMEM((CH,), jnp.int32),
            pltpu.VMEM((CH,), jnp.int32),
            pltpu.VMEM((CH, NF), jnp.float32),
            pltpu.VMEM((CH, NF), jnp.float32),
            pltpu.VMEM((CH,), jnp.float32),
            pltpu.VMEM((CH,), jnp.float32),
            pltpu.VMEM((2, CH, 128), jnp.float32),
            pltpu.VMEM((CH,), jnp.float32),
            pltpu.VMEM((CH,), jnp.float32),
            pltpu.SemaphoreType.DMA,
        ],
    )
    p, mass = run(users, items, song_index, user_factors, item_factors,
                  ub1, ib1, rat_a, rat_b)
    return jnp.stack((p, mass), axis=1)[:, :, None]
